# baseline (speedup 1.0000x reference)
import functools

import jax
import jax.numpy as jnp
from jax import lax
from jax.experimental import pallas as pl
from jax.experimental.pallas import tpu as pltpu
from jax.experimental.pallas import tpu_sc as plsc

N_COND = 26
N_COL = 128
BATCH = 4096
K = N_COND * N_COL
L = 16

NC = 2
NS = 16
NW = NC * NS
ROWS_PER_W = BATCH // NW
R = 2
CHUNKS = ROWS_PER_W // R
CHUNK_ELEMS = R * K
NBUF = 8


def _permute_rows(perm_v, in_v, out_v):
    @plsc.parallel_loop(0, K // L, unroll=4)
    def body(j):
        p = perm_v[pl.ds(j * L, L)]
        q = lax.shift_right_logical(p, 7)
        c = p & 127
        for r in range(R):
            rv = jnp.full((L,), r, jnp.int32)
            g = plsc.load_gather(in_v, [q, rv, c])
            out_v[pl.ds(j * L + r * K, L)] = g


def _sc_kernel(m_hbm, perm_hbm, out_hbm, perm_v, *bufs_and_sems):
    ins = bufs_and_sems[:NBUF]
    outs = bufs_and_sems[NBUF:2 * NBUF]
    sins = bufs_and_sems[2 * NBUF:3 * NBUF]
    souts = bufs_and_sems[3 * NBUF:4 * NBUF]

    wid = lax.axis_index("s") * NC + lax.axis_index("c")
    row0 = wid * ROWS_PER_W
    obase = row0 * K

    def start_in(c, p):
        b = row0 + c * R
        pltpu.async_copy(m_hbm.at[:, pl.ds(b, R), :], ins[p], sins[p])

    def wait_in(p):
        pltpu.make_async_copy(
            m_hbm.at[:, pl.ds(0, R), :], ins[p], sins[p]).wait()

    def start_out(c, p):
        off = obase + c * CHUNK_ELEMS
        pltpu.async_copy(outs[p], out_hbm.at[pl.ds(off, CHUNK_ELEMS)], souts[p])

    def wait_out(p):
        pltpu.make_async_copy(
            outs[p], out_hbm.at[pl.ds(0, CHUNK_ELEMS)], souts[p]).wait()

    for p in range(NBUF):
        start_in(p, p)
    pltpu.sync_copy(perm_hbm, perm_v)

    def step(k, _):
        for p in range(NBUF):
            c = NBUF * k + p
            wait_in(p)
            @pl.when(k > 0)
            def _():
                wait_out(p)
            _permute_rows(perm_v, ins[p], outs[p])
            start_out(c, p)
            @pl.when(c + NBUF < CHUNKS)
            def _():
                start_in(c + NBUF, p)
        return 0

    lax.fori_loop(0, CHUNKS // NBUF, step, 0, unroll=False)
    for p in range(NBUF):
        wait_out(p)


@jax.jit
def _run(m_t, perm):
    mesh = plsc.VectorSubcoreMesh(core_axis_name="c", subcore_axis_name="s")
    f = pl.kernel(
        _sc_kernel,
        out_type=jax.ShapeDtypeStruct((BATCH * K,), jnp.float32),
        mesh=mesh,
        compiler_params=pltpu.CompilerParams(needs_layout_passes=False),
        scratch_types=(
            [pltpu.VMEM((K,), jnp.int32)]
            + [pltpu.VMEM((N_COND, R, N_COL), jnp.float32)] * NBUF
            + [pltpu.VMEM((CHUNK_ELEMS,), jnp.float32)] * NBUF
            + [pltpu.SemaphoreType.DMA] * (2 * NBUF)
        ),
    )
    return f(m_t, perm)


def kernel(M, permutator):
    m_t = jnp.transpose(M, (1, 0, 2))
    perm = permutator.astype(jnp.int32)
    out = _run(m_t, perm)
    return out.reshape(BATCH, K, 1)


# speedup vs baseline: 1.0200x; 1.0200x over previous
import functools

import jax
import jax.numpy as jnp
from jax import lax
from jax.experimental import pallas as pl
from jax.experimental.pallas import tpu as pltpu
from jax.experimental.pallas import tpu_sc as plsc

N_COND = 26
N_COL = 128
BATCH = 4096
K = N_COND * N_COL
L = 16

NC = 2
NS = 16
NW = NC * NS
ROWS_PER_W = BATCH // NW
R = 4
CHUNKS = ROWS_PER_W // R
CHUNK_ELEMS = R * K
NBUF = 4


def _permute_rows(perm_v, in_v, out_v):
    @plsc.parallel_loop(0, K // L, unroll=4)
    def body(j):
        p = perm_v[pl.ds(j * L, L)]
        q = lax.shift_right_logical(p, 7)
        c = p & 127
        for r in range(R):
            rv = jnp.full((L,), r, jnp.int32)
            g = plsc.load_gather(in_v, [q, rv, c])
            out_v[pl.ds(j * L + r * K, L)] = g


def _sc_kernel(m_hbm, perm_hbm, out_hbm, perm_v, *bufs_and_sems):
    ins = bufs_and_sems[:NBUF]
    outs = bufs_and_sems[NBUF:2 * NBUF]
    sins = bufs_and_sems[2 * NBUF:3 * NBUF]
    souts = bufs_and_sems[3 * NBUF:4 * NBUF]

    wid = lax.axis_index("s") * NC + lax.axis_index("c")
    row0 = wid * ROWS_PER_W
    obase = row0 * K

    def start_in(c, p):
        b = row0 + c * R
        pltpu.async_copy(m_hbm.at[:, pl.ds(b, R), :], ins[p], sins[p])

    def wait_in(p):
        pltpu.make_async_copy(
            m_hbm.at[:, pl.ds(0, R), :], ins[p], sins[p]).wait()

    def start_out(c, p):
        off = obase + c * CHUNK_ELEMS
        pltpu.async_copy(outs[p], out_hbm.at[pl.ds(off, CHUNK_ELEMS)], souts[p])

    def wait_out(p):
        pltpu.make_async_copy(
            outs[p], out_hbm.at[pl.ds(0, CHUNK_ELEMS)], souts[p]).wait()

    for p in range(NBUF):
        start_in(p, p)
    pltpu.sync_copy(perm_hbm, perm_v)

    def step(k, _):
        for p in range(NBUF):
            c = NBUF * k + p
            wait_in(p)
            @pl.when(k > 0)
            def _():
                wait_out(p)
            _permute_rows(perm_v, ins[p], outs[p])
            start_out(c, p)
            @pl.when(c + NBUF < CHUNKS)
            def _():
                start_in(c + NBUF, p)
        return 0

    lax.fori_loop(0, CHUNKS // NBUF, step, 0, unroll=False)
    for p in range(NBUF):
        wait_out(p)


@jax.jit
def _run(m_t, perm):
    mesh = plsc.VectorSubcoreMesh(core_axis_name="c", subcore_axis_name="s")
    f = pl.kernel(
        _sc_kernel,
        out_type=jax.ShapeDtypeStruct((BATCH * K,), jnp.float32),
        mesh=mesh,
        compiler_params=pltpu.CompilerParams(needs_layout_passes=False),
        scratch_types=(
            [pltpu.VMEM((K,), jnp.int32)]
            + [pltpu.VMEM((N_COND, R, N_COL), jnp.float32)] * NBUF
            + [pltpu.VMEM((CHUNK_ELEMS,), jnp.float32)] * NBUF
            + [pltpu.SemaphoreType.DMA] * (2 * NBUF)
        ),
    )
    return f(m_t, perm)


def kernel(M, permutator):
    m_t = jnp.transpose(M, (1, 0, 2))
    perm = permutator.astype(jnp.int32)
    out = _run(m_t, perm)
    return out.reshape(BATCH, K, 1)
